# baseline (speedup 1.0000x reference)
import functools

import jax
import jax.numpy as jnp
from jax import lax
from jax.experimental import pallas as pl
from jax.experimental.pallas import tpu as pltpu
from jax.experimental.pallas import tpu_sc as plsc

_INPUT_DIM = 128
_G_DIM = 32
_TAU = 0.95
_K = 8
_N_NB = 32
_BLOCK = 1000

_NW = 32
_EPW = 320
_CH = 16


def _score_body(self_ref, nb_ref, mask_ref, wg_ref, ek_ref, idx_ref, r_ref):
    b = self_ref.shape[0]
    nb = nb_ref[...]
    wg = wg_ref[...]
    g_self = lax.dot_general(
        self_ref[...], wg, (((1,), (1,)), ((), ())),
        preferred_element_type=jnp.float32)
    wg_b = jnp.broadcast_to(wg[None], (b, _G_DIM, _INPUT_DIM))
    gt3 = lax.dot_general(wg_b, nb, (((2,), (2,)), ((0,), (0,))),
                          preferred_element_type=jnp.float32)
    d3 = gt3 - g_self[:, :, None]
    sq = jnp.sum(d3 * d3, axis=1)
    sq_t = sq.T
    mask_t = mask_ref[...].T
    scores = jnp.where(mask_t > 0, jnp.exp(sq_t * (-1.0 / _TAU)), -1e30)

    iota = lax.broadcasted_iota(jnp.int32, (_N_NB, b), 0)
    s = scores
    sel_idx = []
    sel_mval = []
    for _ in range(_K):
        m = jnp.max(s, axis=0, keepdims=True)
        cand = s == m
        idx = jnp.min(jnp.where(cand, iota, _N_NB),
                      axis=0, keepdims=True)
        hit = iota == idx
        sel_idx.append(idx)
        sel_mval.append(jnp.max(jnp.where(hit, mask_t, 0.0),
                                axis=0, keepdims=True))
        s = jnp.where(hit, -jnp.inf, s)

    idx_t = jnp.concatenate(sel_idx, axis=0)
    mval_t = jnp.concatenate(sel_mval, axis=0)
    denom = jnp.maximum(jnp.sum(mval_t, axis=0, keepdims=True), 1e-8)
    r_ref[...] = lax.dot_general(
        (mval_t / denom).T, ek_ref[...], (((1,), (0,)), ((), ())),
        preferred_element_type=jnp.float32)

    ent = pl.program_id(0) * b + lax.broadcasted_iota(jnp.int32, (b, _K), 0)
    idx_ref[...] = ent * _N_NB + idx_t.T


def _tc_score(self_vectors, neighbor_vectors, masks2, W_g):
    n = self_vectors.shape[0]
    grid = (n // _BLOCK,)
    return pl.pallas_call(
        _score_body,
        grid=grid,
        in_specs=[
            pl.BlockSpec((_BLOCK, _INPUT_DIM), lambda i: (i, 0)),
            pl.BlockSpec((_BLOCK, _N_NB, _INPUT_DIM), lambda i: (i, 0, 0)),
            pl.BlockSpec((_BLOCK, _N_NB), lambda i: (i, 0)),
            pl.BlockSpec((_G_DIM, _INPUT_DIM), lambda i: (0, 0)),
            pl.BlockSpec((_K, _K * 16), lambda i: (0, 0)),
        ],
        out_specs=[
            pl.BlockSpec((_BLOCK, _K), lambda i: (i, 0)),
            pl.BlockSpec((_BLOCK, _K * 16), lambda i: (i, 0)),
        ],
        out_shape=[
            jax.ShapeDtypeStruct((n, _K), jnp.int32),
            jax.ShapeDtypeStruct((n, _K * 16), jnp.float32),
        ],
    )(self_vectors, neighbor_vectors, masks2, W_g,
      jnp.repeat(jnp.eye(_K, dtype=jnp.float32), 16, axis=1))


def _sc_aggregate(nb_flat, idx_flat, r_flat, n):
    mesh = plsc.VectorSubcoreMesh(core_axis_name="c", subcore_axis_name="s")

    @functools.partial(
        pl.kernel, mesh=mesh,
        out_type=jax.ShapeDtypeStruct((n, _INPUT_DIM), jnp.float32),
        scratch_types=[
            pltpu.VMEM((_CH * _K,), jnp.int32),
            pltpu.VMEM((_CH, _K * 16), jnp.float32),
            pltpu.VMEM((_CH * _K, _INPUT_DIM), jnp.float32),
            pltpu.VMEM((_CH, _INPUT_DIM), jnp.float32),
            pltpu.SemaphoreType.DMA,
        ],
    )
    def k(nb_hbm, idx_hbm, r_hbm, out_hbm, idx_v, r_v, rows_v, acc_v, sem):
        wid = lax.axis_index("s") * 2 + lax.axis_index("c")
        base_e = wid * _EPW

        def chunk_body(ci, carry):
            e0 = base_e + ci * _CH

            @pl.when(e0 < n)
            def _do():
                pltpu.sync_copy(idx_hbm.at[pl.ds(e0 * _K, _CH * _K)], idx_v)
                pltpu.sync_copy(r_hbm.at[pl.ds(e0, _CH)], r_v)
                pltpu.async_copy(nb_hbm.at[idx_v], rows_v, sem).wait()
                for ei in range(_CH):
                    splats = [r_v[ei, pl.ds(j * 16, 16)] for j in range(_K)]
                    for kk in range(_INPUT_DIM // 16):
                        acc = jnp.zeros((16,), jnp.float32)
                        for j in range(_K):
                            acc = acc + (rows_v[ei * _K + j,
                                                pl.ds(kk * 16, 16)]
                                         * splats[j])
                        acc_v[ei, pl.ds(kk * 16, 16)] = acc
                pltpu.sync_copy(acc_v, out_hbm.at[pl.ds(e0, _CH)])
            return carry

        lax.fori_loop(0, _EPW // _CH, chunk_body, 0)

    return k(nb_flat, idx_flat, r_flat)


@functools.partial(jax.jit, static_argnames=())
def kernel(self_vectors, neighbor_vectors, masks, W_g):
    n = self_vectors.shape[0]
    masks2 = masks.reshape(n, _N_NB)
    idx, r = _tc_score(self_vectors, neighbor_vectors, masks2, W_g)
    nb_flat = neighbor_vectors.reshape(n * _N_NB, _INPUT_DIM)
    return _sc_aggregate(nb_flat, idx.reshape(n * _K), r, n)


# speedup vs baseline: 1.9238x; 1.9238x over previous
import functools

import jax
import jax.numpy as jnp
from jax import lax
from jax.experimental import pallas as pl
from jax.experimental.pallas import tpu as pltpu

_INPUT_DIM = 128
_G_DIM = 32
_TAU = 0.95
_K = 8
_N_NB = 32
_BLOCK = 1000


def _agg_body(self_ref, nb_ref, mask_ref, wg_ref, out_ref):
    b = self_ref.shape[0]
    nb = nb_ref[...]
    wg = wg_ref[...]
    g_self = lax.dot_general(
        self_ref[...], wg, (((1,), (1,)), ((), ())),
        preferred_element_type=jnp.float32)
    g_nb = lax.dot_general(
        nb.reshape(b * _N_NB, _INPUT_DIM), wg, (((1,), (1,)), ((), ())),
        preferred_element_type=jnp.float32)
    d3 = g_nb.reshape(b, _N_NB, _G_DIM) - g_self[:, None, :]
    sq = jnp.sum(d3 * d3, axis=-1)
    sq_t = sq.T
    mask_t = mask_ref[...].T
    scores = jnp.where(mask_t > 0, jnp.exp(sq_t * (-1.0 / _TAU)), -1e30)

    iota = lax.broadcasted_iota(jnp.int32, (_N_NB, b), 0)
    s = scores
    w_t = jnp.zeros((_N_NB, b), jnp.float32)
    for _ in range(_K):
        m = jnp.max(s, axis=0, keepdims=True)
        cand = s == m
        idx = jnp.min(jnp.where(cand, iota, _N_NB),
                      axis=0, keepdims=True)
        hit = iota == idx
        w_t = jnp.where(hit, 1.0, w_t)
        s = jnp.where(hit, -jnp.inf, s)

    wm_t = w_t * mask_t
    recip = 1.0 / jnp.maximum(jnp.sum(wm_t, axis=0), 1e-8)
    w = wm_t.T
    summed = lax.dot_general(
        w[:, None, :], nb, (((2,), (1,)), ((0,), (0,))),
        preferred_element_type=jnp.float32)[:, 0, :]
    out_ref[...] = summed * recip[:, None]


@functools.partial(jax.jit, static_argnames=())
def kernel(self_vectors, neighbor_vectors, masks, W_g):
    n = self_vectors.shape[0]
    masks2 = masks.reshape(n, _N_NB)
    grid = (n // _BLOCK,)
    return pl.pallas_call(
        _agg_body,
        grid=grid,
        in_specs=[
            pl.BlockSpec((_BLOCK, _INPUT_DIM), lambda i: (i, 0)),
            pl.BlockSpec((_BLOCK, _N_NB, _INPUT_DIM), lambda i: (i, 0, 0)),
            pl.BlockSpec((_BLOCK, _N_NB), lambda i: (i, 0)),
            pl.BlockSpec((_G_DIM, _INPUT_DIM), lambda i: (0, 0)),
        ],
        out_specs=pl.BlockSpec((_BLOCK, _INPUT_DIM), lambda i: (i, 0)),
        out_shape=jax.ShapeDtypeStruct((n, _INPUT_DIM), jnp.float32),
    )(self_vectors, neighbor_vectors, masks2, W_g)


# speedup vs baseline: 2.1080x; 1.0957x over previous
import functools

import jax
import jax.numpy as jnp
from jax import lax
from jax.experimental import pallas as pl
from jax.experimental.pallas import tpu as pltpu

_INPUT_DIM = 128
_G_DIM = 32
_TAU = 0.95
_K = 8
_N_NB = 32
_BLOCK = 1000


def _agg_body(self_ref, nb_ref, mask_ref, wg_ref, out_ref):
    b = self_ref.shape[0]
    nb = nb_ref[...]
    wg = wg_ref[...]
    g_self = lax.dot_general(
        self_ref[...], wg, (((1,), (1,)), ((), ())),
        preferred_element_type=jnp.float32)
    wg_b = jnp.broadcast_to(wg[None], (b, _G_DIM, _INPUT_DIM))
    gt3 = lax.dot_general(wg_b, nb, (((2,), (2,)), ((0,), (0,))),
                          preferred_element_type=jnp.float32)
    d3 = gt3 - g_self[:, :, None]
    sq = jnp.sum(d3 * d3, axis=1)
    sq_t = sq.T
    mask_t = mask_ref[...].T
    scores = jnp.where(mask_t > 0, jnp.exp(-sq_t / _TAU), -1e30)

    iota = lax.broadcasted_iota(jnp.int32, (_N_NB, b), 0)
    s = scores
    w_t = jnp.zeros((_N_NB, b), jnp.float32)
    for _ in range(_K):
        m = jnp.max(s, axis=0, keepdims=True)
        cand = s == m
        idx = jnp.min(jnp.where(cand, iota, _N_NB),
                      axis=0, keepdims=True)
        hit = iota == idx
        w_t = jnp.where(hit, 1.0, w_t)
        s = jnp.where(hit, -jnp.inf, s)

    wm_t = w_t * mask_t
    recip = 1.0 / jnp.maximum(jnp.sum(wm_t, axis=0), 1e-8)
    w = wm_t.T
    summed = lax.dot_general(
        w[:, None, :], nb, (((2,), (1,)), ((0,), (0,))),
        preferred_element_type=jnp.float32)[:, 0, :]
    out_ref[...] = summed * recip[:, None]


@functools.partial(jax.jit, static_argnames=())
def kernel(self_vectors, neighbor_vectors, masks, W_g):
    n = self_vectors.shape[0]
    masks2 = masks.reshape(n, _N_NB)
    grid = (n // _BLOCK,)
    return pl.pallas_call(
        _agg_body,
        grid=grid,
        in_specs=[
            pl.BlockSpec((_BLOCK, _INPUT_DIM), lambda i: (i, 0)),
            pl.BlockSpec((_BLOCK, _N_NB, _INPUT_DIM), lambda i: (i, 0, 0)),
            pl.BlockSpec((_BLOCK, _N_NB), lambda i: (i, 0)),
            pl.BlockSpec((_G_DIM, _INPUT_DIM), lambda i: (0, 0)),
        ],
        out_specs=pl.BlockSpec((_BLOCK, _INPUT_DIM), lambda i: (i, 0)),
        out_shape=jax.ShapeDtypeStruct((n, _INPUT_DIM), jnp.float32),
    )(self_vectors, neighbor_vectors, masks2, W_g)


# speedup vs baseline: 2.1252x; 1.0081x over previous
import functools

import jax
import jax.numpy as jnp
from jax import lax
from jax.experimental import pallas as pl
from jax.experimental.pallas import tpu as pltpu

_INPUT_DIM = 128
_G_DIM = 32
_TAU = 0.95
_K = 8
_N_NB = 32
_BLOCK = 1000


def _agg_body(self_ref, nb_ref, mask_ref, wg_ref, out_ref):
    b = self_ref.shape[0]
    nb = nb_ref[...]
    wg = wg_ref[...]
    g_self = lax.dot_general(
        self_ref[...], wg, (((1,), (1,)), ((), ())),
        preferred_element_type=jnp.float32)
    wg_b = jnp.broadcast_to(wg[None], (b, _G_DIM, _INPUT_DIM))
    gt3 = lax.dot_general(wg_b, nb, (((2,), (2,)), ((0,), (0,))),
                          preferred_element_type=jnp.float32)
    d3 = gt3 - g_self[:, :, None]
    sq = jnp.sum(d3 * d3, axis=1)
    sq_t = sq.T
    mask_t = mask_ref[...].T
    scores = jnp.where(mask_t > 0, jnp.exp(sq_t * (-1.0 / _TAU)), -1e30)

    iota = lax.broadcasted_iota(jnp.int32, (_N_NB, b), 0)
    s = scores
    w_t = jnp.zeros((_N_NB, b), jnp.float32)
    for _ in range(_K):
        m = jnp.max(s, axis=0, keepdims=True)
        cand = s == m
        idx = jnp.min(jnp.where(cand, iota, _N_NB),
                      axis=0, keepdims=True)
        hit = iota == idx
        w_t = jnp.where(hit, 1.0, w_t)
        s = jnp.where(hit, -jnp.inf, s)

    wm_t = w_t * mask_t
    recip = 1.0 / jnp.maximum(jnp.sum(wm_t, axis=0), 1e-8)
    w = wm_t.T
    summed = lax.dot_general(
        w[:, None, :], nb, (((2,), (1,)), ((0,), (0,))),
        preferred_element_type=jnp.float32)[:, 0, :]
    out_ref[...] = summed * recip[:, None]


@functools.partial(jax.jit, static_argnames=())
def kernel(self_vectors, neighbor_vectors, masks, W_g):
    n = self_vectors.shape[0]
    masks2 = masks.reshape(n, _N_NB)
    grid = (n // _BLOCK,)
    return pl.pallas_call(
        _agg_body,
        grid=grid,
        in_specs=[
            pl.BlockSpec((_BLOCK, _INPUT_DIM), lambda i: (i, 0)),
            pl.BlockSpec((_BLOCK, _N_NB, _INPUT_DIM), lambda i: (i, 0, 0)),
            pl.BlockSpec((_BLOCK, _N_NB), lambda i: (i, 0)),
            pl.BlockSpec((_G_DIM, _INPUT_DIM), lambda i: (0, 0)),
        ],
        out_specs=pl.BlockSpec((_BLOCK, _INPUT_DIM), lambda i: (i, 0)),
        out_shape=jax.ShapeDtypeStruct((n, _INPUT_DIM), jnp.float32),
    )(self_vectors, neighbor_vectors, masks2, W_g)


# speedup vs baseline: 2.2133x; 1.0415x over previous
import functools

import jax
import jax.numpy as jnp
from jax import lax
from jax.experimental import pallas as pl
from jax.experimental.pallas import tpu as pltpu

_INPUT_DIM = 128
_G_DIM = 32
_TAU = 0.95
_K = 8
_N_NB = 32
_BLOCK = 1000


def _agg_body(self_ref, nb_ref, mask_ref, wg_ref, pad_ref, out_ref):
    b = self_ref.shape[0]
    nb = nb_ref[...]
    wg = wg_ref[...]
    g_self = lax.dot_general(
        self_ref[...], wg, (((1,), (1,)), ((), ())),
        preferred_element_type=jnp.float32)
    wg_b = jnp.broadcast_to(wg[None], (b, _G_DIM, _INPUT_DIM))
    gt3 = lax.dot_general(wg_b, nb, (((2,), (2,)), ((0,), (0,))),
                          preferred_element_type=jnp.float32)
    d3 = gt3 - g_self[:, :, None]
    sq = jnp.sum(d3 * d3, axis=1)
    sq_t = sq.T
    mask_t = mask_ref[...].T
    scores = jnp.where(mask_t > 0, jnp.exp(sq_t * (-1.0 / _TAU)), -1e30)

    iota = lax.broadcasted_iota(jnp.int32, (_N_NB, b), 0)
    s = scores
    w_t = jnp.zeros((_N_NB, b), jnp.float32)
    for _ in range(_K):
        m = jnp.max(s, axis=0, keepdims=True)
        cand = s == m
        idx = jnp.min(jnp.where(cand, iota, _N_NB),
                      axis=0, keepdims=True)
        hit = iota == idx
        w_t = jnp.where(hit, 1.0, w_t)
        s = jnp.where(hit, -jnp.inf, s)

    wm_t = w_t * mask_t
    recip = 1.0 / jnp.maximum(jnp.sum(wm_t, axis=0), 1e-8)
    w = wm_t.T
    summed = lax.dot_general(
        w[:, None, :], nb, (((2,), (1,)), ((0,), (0,))),
        preferred_element_type=jnp.float32)[:, 0, :]
    out_ref[...] = summed * recip[:, None]


@functools.partial(jax.jit, static_argnames=())
def kernel(self_vectors, neighbor_vectors, masks, W_g):
    n = self_vectors.shape[0]
    masks2 = masks.reshape(n, _N_NB)
    pad = jnp.zeros((64, _INPUT_DIM), jnp.float32)
    grid = (n // _BLOCK,)
    return pl.pallas_call(
        _agg_body,
        grid=grid,
        in_specs=[
            pl.BlockSpec((_BLOCK, _INPUT_DIM), lambda i: (i, 0)),
            pl.BlockSpec((_BLOCK, _N_NB, _INPUT_DIM), lambda i: (i, 0, 0)),
            pl.BlockSpec((_BLOCK, _N_NB), lambda i: (i, 0)),
            pl.BlockSpec((_G_DIM, _INPUT_DIM), lambda i: (0, 0)),
            pl.BlockSpec((64, _INPUT_DIM), lambda i: (0, 0)),
        ],
        out_specs=pl.BlockSpec((_BLOCK, _INPUT_DIM), lambda i: (i, 0)),
        out_shape=jax.ShapeDtypeStruct((n, _INPUT_DIM), jnp.float32),
    )(self_vectors, neighbor_vectors, masks2, W_g, pad)
